# baseline (speedup 1.0000x reference)
import functools

import jax
import jax.numpy as jnp
from jax import lax
from jax.experimental import pallas as pl
from jax.experimental.pallas import tpu as pltpu
from jax.experimental.pallas import tpu_sc as plsc

N = 10000
E = 320000
D = 128
NOISE = 64
DH = 128
DZ = 64
S = 2

NC = 2
NS = 16
CH = 128
NCHUNK = E // CH
NPAD = 10240
NPT = NPAD // NS
CPW = 80
E_PAD = NC * NS * CPW * CH

def _offset_idx(idx_ref, off):
    for j in range(CH // 16):
        sl = pl.ds(j * 16, 16)
        idx_ref[sl] = idx_ref[sl] + off


def _zero_acc(zx, acc, sub):
    pltpu.sync_copy(zx, acc.at[pl.ds(sub * NPT, NPT)])
    plsc.subcore_barrier()


def _writeout(acc, out_ref, core, sub):
    plsc.subcore_barrier()
    pltpu.sync_copy(acc.at[pl.ds(sub * NPT, NPT)],
                    out_ref.at[pl.ds(core * NPAD + sub * NPT, NPT)])


def _accum_pass(table, dstp, srcf, dst_i, rows0, rows1, gs0, gs1, ds0, ds1,
                acc, start):
    pltpu.async_copy(dstp.at[pl.ds(start * CH, CH)], dst_i.at[0], ds0)
    pltpu.async_copy(dstp.at[pl.ds((start + 1) * CH, CH)], dst_i.at[1], ds1)
    pltpu.async_copy(table.at[srcf.at[pl.ds(0, CH)]], rows0, gs0)

    def pair(p, carry):
        for b, (rows, gs, dsem), (orows, ogs) in (
                (0, (rows0, gs0, ds0), (rows1, gs1)),
                (1, (rows1, gs1, ds1), (rows0, gs0))):
            k = 2 * p + b
            di = dst_i.at[b]
            pltpu.make_async_copy(table.at[pl.ds(0, CH)], rows, gs).wait()
            pltpu.make_async_copy(dstp.at[pl.ds(0, CH)], di, dsem).wait()
            nk = jnp.minimum(k + 1, CPW - 1)
            pltpu.async_copy(table.at[srcf.at[pl.ds(nk * CH, CH)]], orows, ogs)
            pltpu.sync_copy(rows, acc.at[di], add=True)
            pk = (start + jnp.minimum(k + 2, CPW - 1)) * CH
            pltpu.async_copy(dstp.at[pl.ds(pk, CH)], di, dsem)
        return carry

    lax.fori_loop(0, CPW // 2, pair, 0)
    pltpu.make_async_copy(table.at[pl.ds(0, CH)], rows0, gs0).wait()
    pltpu.make_async_copy(dstp.at[pl.ds(0, CH)], dst_i.at[0], ds0).wait()
    pltpu.make_async_copy(dstp.at[pl.ds(0, CH)], dst_i.at[1], ds1).wait()


def _load_src(srcp, srcf, start, row_off):
    pltpu.sync_copy(srcp.at[pl.ds(start * CH, CPW * CH)], srcf)

    def add_off(i, carry):
        sl = pl.ds(i * 16, 16)
        srcf[sl] = srcf[sl] + row_off
        return carry

    lax.fori_loop(0, CPW * CH // 16, add_off, 0)


def _sc_agg1_body(feat, eps_pair, srcp, dstp, zx, aggx_out, aggep_out,
                  srcf, dst_i, rows0, rows1, gs0, gs1, ds0, ds1, acc):
    core = lax.axis_index("c")
    sub = lax.axis_index("s")
    wid = sub * NC + core
    start = wid * CPW
    pltpu.sync_copy(srcp.at[pl.ds(start * CH, CPW * CH)], srcf)
    bufs = (srcf, dst_i, rows0, rows1, gs0, gs1, ds0, ds1, acc)
    _zero_acc(zx, acc, sub)
    _accum_pass(feat, dstp, *bufs, start)
    _writeout(acc, aggx_out, core, sub)
    _zero_acc(zx, acc, sub)
    _accum_pass(eps_pair, dstp, *bufs, start)
    _writeout(acc, aggep_out, core, sub)


def _sc_agg2_body(h2, srcp, dstp, zx, agg_out,
                  srcf, dst_i, rows0, rows1, gs0, gs1, ds0, ds1, acc):
    core = lax.axis_index("c")
    sub = lax.axis_index("s")
    bufs = (srcf, dst_i, rows0, rows1, gs0, gs1, ds0, ds1, acc)
    _zero_acc(zx, acc, sub)
    for half in (0, 1):
        start = (2 * sub + half) * CPW
        _load_src(srcp, srcf, start, core * N)
        _accum_pass(h2, dstp, *bufs, start)
    _writeout(acc, agg_out, core, sub)


@functools.lru_cache(maxsize=None)
def _sc_kernels():
    mesh = plsc.VectorSubcoreMesh(core_axis_name="c", subcore_axis_name="s",
                                  num_cores=NC, num_subcores=NS)
    scratch = [
        pltpu.VMEM((CPW * CH,), jnp.int32),
        pltpu.VMEM((2, CH), jnp.int32),
        pltpu.VMEM((CH, D), jnp.float32),
        pltpu.VMEM((CH, D), jnp.float32),
        pltpu.SemaphoreType.DMA,
        pltpu.SemaphoreType.DMA,
        pltpu.SemaphoreType.DMA,
        pltpu.SemaphoreType.DMA,
        pltpu.VMEM_SHARED((NPAD, D), jnp.float32),
    ]
    agg1 = pl.kernel(
        _sc_agg1_body,
        out_type=[
            jax.ShapeDtypeStruct((NC * NPAD, D), jnp.float32),
            jax.ShapeDtypeStruct((NC * NPAD, D), jnp.float32),
        ],
        mesh=mesh,
        scratch_types=list(scratch),
    )
    agg2 = pl.kernel(
        _sc_agg2_body,
        out_type=jax.ShapeDtypeStruct((S * NPAD, DH), jnp.float32),
        mesh=mesh,
        scratch_types=list(scratch),
    )
    return agg1, agg2


BM = 1000


def _layer1_body(scal, x_ref, aggx_ref, eps_ref, aggep_ref, wx_ref, we_ref,
                 bu_ref, h_ref):
    s = pl.program_id(0)
    su = scal[0, 0]
    a = su * x_ref[...] + aggx_ref[0] + aggx_ref[1]
    ep = aggep_ref[0] + aggep_ref[1]
    e = su * eps_ref[0] + jnp.where(s == 0, ep[:, :NOISE], ep[:, NOISE:])
    h = jnp.dot(a, wx_ref[...], preferred_element_type=jnp.float32)
    h += jnp.dot(e, we_ref[...], preferred_element_type=jnp.float32)
    h += bu_ref[...]
    h_ref[0] = jnp.maximum(h, 0.0)


def _layer2_body(scal, h_ref, agg_ref, wmu_ref, wsig_ref, bmu_ref, bsig_ref,
                 z_ref):
    smu = scal[0, 1]
    ssig = scal[0, 2]
    c = scal[0, 3]
    h = h_ref[0]
    agg = agg_ref[0]
    mu = jnp.dot(smu * h + agg, wmu_ref[...], preferred_element_type=jnp.float32)
    mu += bmu_ref[...]
    ls = jnp.dot(ssig * h + agg, wsig_ref[...], preferred_element_type=jnp.float32)
    ls += bsig_ref[...]
    z_ref[0] = mu + c * jnp.exp(ls * 0.5)


def _decoder_body(zr_ref, zc_ref, out_ref):
    logits = lax.dot_general(zr_ref[0], zc_ref[0], (((1,), (1,)), ((), ())),
                             preferred_element_type=jnp.float32)
    out_ref[0] = jax.nn.sigmoid(logits)


def _layer1(scal, feat, aggx, epsilon, aggep, wx, we, bu):
    return pl.pallas_call(
        _layer1_body,
        grid=(S, N // BM),
        in_specs=[
            pl.BlockSpec((1, 4), lambda s, i: (0, 0)),
            pl.BlockSpec((BM, D), lambda s, i: (i, 0)),
            pl.BlockSpec((NC, BM, D), lambda s, i: (0, i, 0)),
            pl.BlockSpec((1, BM, NOISE), lambda s, i: (s, i, 0)),
            pl.BlockSpec((NC, BM, D), lambda s, i: (0, i, 0)),
            pl.BlockSpec((D, DH), lambda s, i: (0, 0)),
            pl.BlockSpec((NOISE, DH), lambda s, i: (0, 0)),
            pl.BlockSpec((1, DH), lambda s, i: (0, 0)),
        ],
        out_specs=pl.BlockSpec((1, BM, DH), lambda s, i: (s, i, 0)),
        out_shape=jax.ShapeDtypeStruct((S, N, DH), jnp.float32),
    )(scal, feat, aggx, epsilon, aggep, wx, we, bu)


def _layer2(scal, h, aggh, wmu, wsig, bmu, bsig):
    return pl.pallas_call(
        _layer2_body,
        grid=(S, N // BM),
        in_specs=[
            pl.BlockSpec((1, 4), lambda s, i: (0, 0)),
            pl.BlockSpec((1, BM, DH), lambda s, i: (s, i, 0)),
            pl.BlockSpec((1, BM, DH), lambda s, i: (s, i, 0)),
            pl.BlockSpec((DH, DZ), lambda s, i: (0, 0)),
            pl.BlockSpec((DH, DZ), lambda s, i: (0, 0)),
            pl.BlockSpec((1, DZ), lambda s, i: (0, 0)),
            pl.BlockSpec((1, DZ), lambda s, i: (0, 0)),
        ],
        out_specs=pl.BlockSpec((1, BM, DZ), lambda s, i: (s, i, 0)),
        out_shape=jax.ShapeDtypeStruct((S, N, DZ), jnp.float32),
    )(scal, h, aggh, wmu, wsig, bmu, bsig)


BN = 1024


def _decoder(z):
    return pl.pallas_call(
        _decoder_body,
        grid=(S, N // BM, pl.cdiv(N, BN)),
        in_specs=[
            pl.BlockSpec((1, BM, DZ), lambda s, i, j: (s, i, 0)),
            pl.BlockSpec((1, BN, DZ), lambda s, i, j: (s, j, 0)),
        ],
        out_specs=pl.BlockSpec((1, BM, BN), lambda s, i, j: (s, i, j)),
        out_shape=jax.ShapeDtypeStruct((S, N, N), jnp.float32),
    )(z, z)


def kernel(adj_matrix, feat_matrix, epsilon, normal_sample, Wu, bu, eps_u,
           Wmu, bmu, eps_mu, Wsig, bsig, eps_sig):
    src = adj_matrix[0]
    dst = adj_matrix[1]
    eps_pair = jnp.concatenate([epsilon[0], epsilon[1]], axis=1)
    scal = jnp.stack([1.0 + eps_u, 1.0 + eps_mu, 1.0 + eps_sig,
                      normal_sample[0]]).reshape(1, 4)
    zx = jnp.zeros((NPT, D), jnp.float32)

    srcp = jnp.concatenate([src, jnp.zeros((E_PAD - E,), jnp.int32)])
    pad_dst = N + jnp.arange(E_PAD - E, dtype=jnp.int32) % (NPAD - N)
    dstp = jnp.concatenate([dst, pad_dst])

    sc_agg1, sc_agg2 = _sc_kernels()
    aggx2, aggep2 = sc_agg1(feat_matrix, eps_pair, srcp, dstp, zx)
    h = _layer1(scal, feat_matrix, aggx2.reshape(NC, NPAD, D),
                epsilon, aggep2.reshape(NC, NPAD, D),
                Wu[:D], Wu[D:], bu.reshape(1, DH))
    aggh2 = sc_agg2(h.reshape(S * N, DH), srcp, dstp, zx)
    z = _layer2(scal, h, aggh2.reshape(S, NPAD, DH),
                Wmu, Wsig, bmu.reshape(1, DZ), bsig.reshape(1, DZ))
    prob = _decoder(z)
    return (z, prob)


# speedup vs baseline: 1.5544x; 1.5544x over previous
import functools

import jax
import jax.numpy as jnp
from jax import lax
from jax.experimental import pallas as pl
from jax.experimental.pallas import tpu as pltpu
from jax.experimental.pallas import tpu_sc as plsc

N = 10000
E = 320000
D = 128
NOISE = 64
DH = 128
DZ = 64
S = 2

NC = 2
NS = 16
CH = 128
NCHUNK = E // CH
NPAD = 10240
NPT = NPAD // NS

def _offset_idx(idx_ref, off):
    for j in range(CH // 16):
        sl = pl.ds(j * 16, 16)
        idx_ref[sl] = idx_ref[sl] + off


def _zero_acc(zx, acc, sub):
    pltpu.sync_copy(zx, acc.at[pl.ds(sub * NPT, NPT)])
    plsc.subcore_barrier()


def _writeout(acc, out_ref, core, sub):
    plsc.subcore_barrier()
    pltpu.sync_copy(acc.at[pl.ds(sub * NPT, NPT)],
                    out_ref.at[pl.ds(core * NPAD + sub * NPT, NPT)])


def _accum(table, srcp, dstp, src_i, dst_i, rows, sem, acc,
           first, stride, row_off):
    n = (NCHUNK - first + stride - 1) // stride

    def body(k, carry):
        off = (first + k * stride) * CH
        pltpu.sync_copy(srcp.at[pl.ds(off, CH)], src_i)
        pltpu.sync_copy(dstp.at[pl.ds(off, CH)], dst_i.at[0])
        if row_off is not None:
            for j in range(CH // 16):
                sl = pl.ds(j * 16, 16)
                src_i[sl] = src_i[sl] + row_off
        pltpu.async_copy(table.at[src_i], rows, sem).wait()
        pltpu.sync_copy(rows, acc.at[dst_i.at[0]], add=True)
        return carry

    lax.fori_loop(0, n, body, 0)


def _sc_agg1_body(feat, eps_pair, srcp, dstp, zx, aggx_out, aggep_out,
                  src_i, dst_i, rows, sem, acc):
    core = lax.axis_index("c")
    sub = lax.axis_index("s")
    wid = sub * NC + core
    _zero_acc(zx, acc, sub)
    _accum(feat, srcp, dstp, src_i, dst_i, rows, sem, acc, wid, NC * NS, None)
    _writeout(acc, aggx_out, core, sub)
    _zero_acc(zx, acc, sub)
    _accum(eps_pair, srcp, dstp, src_i, dst_i, rows, sem, acc, wid, NC * NS, None)
    _writeout(acc, aggep_out, core, sub)


def _sc_agg2_body(h2, srcp, dstp, zx, agg_out, src_i, dst_i, rows, sem, acc):
    core = lax.axis_index("c")
    sub = lax.axis_index("s")
    _zero_acc(zx, acc, sub)
    _accum(h2, srcp, dstp, src_i, dst_i, rows, sem, acc, sub, NS, core * N)
    _writeout(acc, agg_out, core, sub)


@functools.lru_cache(maxsize=None)
def _sc_kernels():
    mesh = plsc.VectorSubcoreMesh(core_axis_name="c", subcore_axis_name="s",
                                  num_cores=NC, num_subcores=NS)
    scratch = [
        pltpu.VMEM((CH,), jnp.int32),
        pltpu.VMEM((1, CH), jnp.int32),
        pltpu.VMEM((CH, D), jnp.float32),
        pltpu.SemaphoreType.DMA,
        pltpu.VMEM_SHARED((NPAD, D), jnp.float32),
    ]
    agg1 = pl.kernel(
        _sc_agg1_body,
        out_type=[
            jax.ShapeDtypeStruct((NC * NPAD, D), jnp.float32),
            jax.ShapeDtypeStruct((NC * NPAD, D), jnp.float32),
        ],
        mesh=mesh,
        scratch_types=list(scratch),
    )
    agg2 = pl.kernel(
        _sc_agg2_body,
        out_type=jax.ShapeDtypeStruct((S * NPAD, DH), jnp.float32),
        mesh=mesh,
        scratch_types=list(scratch),
    )
    return agg1, agg2


BM = 1000


def _layer1_body(scal, x_ref, aggx_ref, eps_ref, aggep_ref, wx_ref, we_ref,
                 bu_ref, h_ref):
    s = pl.program_id(0)
    su = scal[0, 0]
    a = su * x_ref[...] + aggx_ref[0] + aggx_ref[1]
    ep = aggep_ref[0] + aggep_ref[1]
    e = su * eps_ref[0] + jnp.where(s == 0, ep[:, :NOISE], ep[:, NOISE:])
    h = jnp.dot(a, wx_ref[...], preferred_element_type=jnp.float32)
    h += jnp.dot(e, we_ref[...], preferred_element_type=jnp.float32)
    h += bu_ref[...]
    h_ref[0] = jnp.maximum(h, 0.0)


def _layer2_body(scal, h_ref, agg_ref, wmu_ref, wsig_ref, bmu_ref, bsig_ref,
                 z_ref):
    smu = scal[0, 1]
    ssig = scal[0, 2]
    c = scal[0, 3]
    h = h_ref[0]
    agg = agg_ref[0]
    mu = jnp.dot(smu * h + agg, wmu_ref[...], preferred_element_type=jnp.float32)
    mu += bmu_ref[...]
    ls = jnp.dot(ssig * h + agg, wsig_ref[...], preferred_element_type=jnp.float32)
    ls += bsig_ref[...]
    z_ref[0] = mu + c * jnp.exp(ls * 0.5)


def _decoder_body(zr_ref, zc_ref, out_ref):
    logits = lax.dot_general(zr_ref[0], zc_ref[0], (((1,), (1,)), ((), ())),
                             preferred_element_type=jnp.float32)
    out_ref[0] = jax.nn.sigmoid(logits)


def _layer1(scal, feat, aggx, epsilon, aggep, wx, we, bu):
    return pl.pallas_call(
        _layer1_body,
        grid=(S, N // BM),
        in_specs=[
            pl.BlockSpec((1, 4), lambda s, i: (0, 0)),
            pl.BlockSpec((BM, D), lambda s, i: (i, 0)),
            pl.BlockSpec((NC, BM, D), lambda s, i: (0, i, 0)),
            pl.BlockSpec((1, BM, NOISE), lambda s, i: (s, i, 0)),
            pl.BlockSpec((NC, BM, D), lambda s, i: (0, i, 0)),
            pl.BlockSpec((D, DH), lambda s, i: (0, 0)),
            pl.BlockSpec((NOISE, DH), lambda s, i: (0, 0)),
            pl.BlockSpec((1, DH), lambda s, i: (0, 0)),
        ],
        out_specs=pl.BlockSpec((1, BM, DH), lambda s, i: (s, i, 0)),
        out_shape=jax.ShapeDtypeStruct((S, N, DH), jnp.float32),
    )(scal, feat, aggx, epsilon, aggep, wx, we, bu)


def _layer2(scal, h, aggh, wmu, wsig, bmu, bsig):
    return pl.pallas_call(
        _layer2_body,
        grid=(S, N // BM),
        in_specs=[
            pl.BlockSpec((1, 4), lambda s, i: (0, 0)),
            pl.BlockSpec((1, BM, DH), lambda s, i: (s, i, 0)),
            pl.BlockSpec((1, BM, DH), lambda s, i: (s, i, 0)),
            pl.BlockSpec((DH, DZ), lambda s, i: (0, 0)),
            pl.BlockSpec((DH, DZ), lambda s, i: (0, 0)),
            pl.BlockSpec((1, DZ), lambda s, i: (0, 0)),
            pl.BlockSpec((1, DZ), lambda s, i: (0, 0)),
        ],
        out_specs=pl.BlockSpec((1, BM, DZ), lambda s, i: (s, i, 0)),
        out_shape=jax.ShapeDtypeStruct((S, N, DZ), jnp.float32),
    )(scal, h, aggh, wmu, wsig, bmu, bsig)


BN = 1024


def _decoder(z):
    return pl.pallas_call(
        _decoder_body,
        grid=(S, N // BM, pl.cdiv(N, BN)),
        in_specs=[
            pl.BlockSpec((1, BM, DZ), lambda s, i, j: (s, i, 0)),
            pl.BlockSpec((1, BN, DZ), lambda s, i, j: (s, j, 0)),
        ],
        out_specs=pl.BlockSpec((1, BM, BN), lambda s, i, j: (s, i, j)),
        out_shape=jax.ShapeDtypeStruct((S, N, N), jnp.float32),
    )(z, z)


def kernel(adj_matrix, feat_matrix, epsilon, normal_sample, Wu, bu, eps_u,
           Wmu, bmu, eps_mu, Wsig, bsig, eps_sig):
    src = adj_matrix[0]
    dst = adj_matrix[1]
    eps_pair = jnp.concatenate([epsilon[0], epsilon[1]], axis=1)
    scal = jnp.stack([1.0 + eps_u, 1.0 + eps_mu, 1.0 + eps_sig,
                      normal_sample[0]]).reshape(1, 4)
    zx = jnp.zeros((NPT, D), jnp.float32)

    sc_agg1, sc_agg2 = _sc_kernels()
    aggx2, aggep2 = sc_agg1(feat_matrix, eps_pair, src, dst, zx)
    h = _layer1(scal, feat_matrix, aggx2.reshape(NC, NPAD, D),
                epsilon, aggep2.reshape(NC, NPAD, D),
                Wu[:D], Wu[D:], bu.reshape(1, DH))
    aggh2 = sc_agg2(h.reshape(S * N, DH), src, dst, zx)
    z = _layer2(scal, h, aggh2.reshape(S, NPAD, DH),
                Wmu, Wsig, bmu.reshape(1, DZ), bsig.reshape(1, DZ))
    prob = _decoder(z)
    return (z, prob)
